# baseline (speedup 1.0000x reference)
import functools

import jax
import jax.numpy as jnp
from jax.experimental import pallas as pl
from jax.experimental.pallas import tpu as pltpu


def _pool_matmul_kernel(x_ref, pw_ref, o_ref):
    o_ref[...] = jnp.dot(
        x_ref[...], pw_ref[...], preferred_element_type=jnp.float32
    ).astype(o_ref.dtype)


def _avg_pool(x, d):
    B, C, H, W = x.shape
    Ho, Wo = H // d, W // d
    if H != Ho * d or W != Wo * d:
        x = x[:, :, : Ho * d, : Wo * d]
        H, W = Ho * d, Wo * d
    M = B * C * Ho
    K = d * W

    a = x.reshape(M, K)

    k = jnp.arange(K)
    pw = ((k % W) // d)[:, None] == jnp.arange(Wo)[None, :]
    pw = pw.astype(jnp.float32) * (1.0 / (d * d))

    tm = 2048
    while M % tm and tm > 8:
        tm //= 2
    grid = (pl.cdiv(M, tm),)

    itemsize = x.dtype.itemsize
    cost = pl.CostEstimate(
        flops=2 * M * K * Wo,
        transcendentals=0,
        bytes_accessed=M * K * itemsize + K * Wo * 4 + M * Wo * itemsize,
    )

    out = pl.pallas_call(
        _pool_matmul_kernel,
        out_shape=jax.ShapeDtypeStruct((M, Wo), x.dtype),
        grid=grid,
        in_specs=[
            pl.BlockSpec((tm, K), lambda i: (i, 0)),
            pl.BlockSpec((K, Wo), lambda i: (0, 0)),
        ],
        out_specs=pl.BlockSpec((tm, Wo), lambda i: (i, 0)),
        compiler_params=pltpu.CompilerParams(
            dimension_semantics=("parallel",),
            vmem_limit_bytes=64 << 20,
        ),
        cost_estimate=cost,
    )(a, pw)

    return out.reshape(B, C, Ho, Wo)


def kernel(x):
    return _avg_pool(x, 2)


# speedup vs baseline: 1.1163x; 1.1163x over previous
import functools

import jax
import jax.numpy as jnp
from jax.experimental import pallas as pl
from jax.experimental.pallas import tpu as pltpu


def _pool_matmul_kernel(x_ref, pw_ref, o_ref):
    o_ref[...] = jnp.dot(
        x_ref[...], pw_ref[...], preferred_element_type=jnp.float32
    ).astype(o_ref.dtype)


def _avg_pool(x, d):
    B, C, H, W = x.shape
    Ho, Wo = H // d, W // d
    if H != Ho * d or W != Wo * d:
        x = x[:, :, : Ho * d, : Wo * d]
        H, W = Ho * d, Wo * d
    M = B * C * Ho
    K = d * W

    a = x.reshape(M, K)

    k = jnp.arange(K)
    pw = ((k % W) // d)[:, None] == jnp.arange(Wo)[None, :]
    pw = pw.astype(jnp.float32) * (1.0 / (d * d))

    tm = 8192
    while M % tm and tm > 8:
        tm //= 2
    grid = (pl.cdiv(M, tm),)

    itemsize = x.dtype.itemsize
    cost = pl.CostEstimate(
        flops=2 * M * K * Wo,
        transcendentals=0,
        bytes_accessed=M * K * itemsize + K * Wo * 4 + M * Wo * itemsize,
    )

    out = pl.pallas_call(
        _pool_matmul_kernel,
        out_shape=jax.ShapeDtypeStruct((M, Wo), x.dtype),
        grid=grid,
        in_specs=[
            pl.BlockSpec((tm, K), lambda i: (i, 0)),
            pl.BlockSpec((K, Wo), lambda i: (0, 0)),
        ],
        out_specs=pl.BlockSpec((tm, Wo), lambda i: (i, 0)),
        compiler_params=pltpu.CompilerParams(
            dimension_semantics=("parallel",),
            vmem_limit_bytes=64 << 20,
        ),
        cost_estimate=cost,
    )(a, pw)

    return out.reshape(B, C, Ho, Wo)


def kernel(x):
    return _avg_pool(x, 2)
